# Initial kernel scaffold; baseline (speedup 1.0000x reference)
import jax
import jax.numpy as jnp
from jax.experimental import pallas as pl


def kernel(x_bk_c, edge_index_batched, W, gamma, beta):
    raise NotImplementedError("write your pallas kernel here")


# speedup vs baseline: 6.7026x; 6.7026x over previous
import functools

import jax
import jax.numpy as jnp
from jax import lax
from jax.experimental import pallas as pl
from jax.experimental.pallas import tpu as pltpu
from jax.experimental.pallas import tpu_sc as plsc

N = 10000
E = 320000
C = 128
BN_EPS = 1e-5

NC, NS, L = 2, 16, 16
NW = NC * NS
EPW = E // NW
K = 80
CHUNKS = EPW // K
NBUF = 2
NAB = 2
NPAD = 10240
ROWS_PER_TILE = NPAD // NS
G = C // L

_mesh = plsc.VectorSubcoreMesh(core_axis_name="c", subcore_axis_name="s")


def _wid():
    return lax.axis_index("s") * NC + lax.axis_index("c")


def _pq_body(x_ref, w_ref, p_ref, q_ref):
    wt = w_ref[0:C, :]
    wb = w_ref[C : 2 * C, :]
    xb = x_ref[...]
    p_ref[...] = jnp.dot(xb, wt - wb, preferred_element_type=jnp.float32)
    q_ref[...] = jnp.dot(xb, wb, preferred_element_type=jnp.float32)


def _make_pq(x, w):
    blk = 1000
    return pl.pallas_call(
        _pq_body,
        grid=(N // blk,),
        in_specs=[
            pl.BlockSpec((blk, C), lambda i: (i, 0)),
            pl.BlockSpec((2 * C, C), lambda i: (0, 0)),
        ],
        out_specs=[
            pl.BlockSpec((blk, C), lambda i: (i, 0)),
            pl.BlockSpec((blk, C), lambda i: (i, 0)),
        ],
        out_shape=[
            jax.ShapeDtypeStruct((N, C), jnp.float32),
            jax.ShapeDtypeStruct((N, C), jnp.float32),
        ],
    )(x, w)


def _ssfold_body(p_ref, q_ref, ss_ref, p2_ref, q2_ref):
    scale = ss_ref[0:1, :]
    shift = ss_ref[1:2, :]
    p2_ref[...] = p_ref[...] * scale
    q2_ref[...] = q_ref[...] * scale + shift


def _ssfold(p, q, ss):
    blk = 1000
    return pl.pallas_call(
        _ssfold_body,
        grid=(N // blk,),
        in_specs=[
            pl.BlockSpec((blk, C), lambda i: (i, 0)),
            pl.BlockSpec((blk, C), lambda i: (i, 0)),
            pl.BlockSpec((2, C), lambda i: (0, 0)),
        ],
        out_specs=[
            pl.BlockSpec((blk, C), lambda i: (i, 0)),
            pl.BlockSpec((blk, C), lambda i: (i, 0)),
        ],
        out_shape=[
            jax.ShapeDtypeStruct((N, C), jnp.float32),
            jax.ShapeDtypeStruct((N, C), jnp.float32),
        ],
    )(p, q, ss)


@functools.partial(
    pl.kernel,
    mesh=_mesh,
    out_type=jax.ShapeDtypeStruct((NW, 2, C), jnp.float32),
    scratch_types=[
        pltpu.VMEM((CHUNKS, K), jnp.int32),
        pltpu.VMEM((CHUNKS, K), jnp.int32),
        pltpu.VMEM((NBUF, K, C), jnp.float32),
        pltpu.VMEM((NBUF, K, C), jnp.float32),
        pltpu.VMEM((2, C), jnp.float32),
        pltpu.SemaphoreType.DMA,
    ],
)
def _stats_kernel(row_hbm, col_hbm, p_hbm, q_hbm, out_hbm,
                  idx_r, idx_c, buf_p, buf_q, stat_v, sem_g):
    wid = _wid()
    pltpu.sync_copy(row_hbm.at[wid], idx_r)
    pltpu.sync_copy(col_hbm.at[wid], idx_c)

    def issue(c, b):
        pltpu.async_copy(p_hbm.at[idx_r.at[c]], buf_p.at[b], sem_g)
        pltpu.async_copy(q_hbm.at[idx_c.at[c]], buf_q.at[b], sem_g)

    def drain(b):
        pltpu.make_async_copy(p_hbm.at[idx_r.at[0]], buf_p.at[b], sem_g).wait()
        pltpu.make_async_copy(q_hbm.at[idx_c.at[0]], buf_q.at[b], sem_g).wait()

    for b in range(NBUF):
        issue(b, b)

    def make_edge(b):
        def edge(j, accs):
            out = []
            for g in range(G):
                sl = pl.ds(g * L, L)
                h = buf_p[b, j, sl] + buf_q[b, j, sl]
                out.append(accs[g] + h)
                out.append(accs[G + g] + h * h)
            return tuple(out[0::2] + out[1::2])

        return edge

    def outer(i, accs):
        for b in range(NBUF):
            c = i * NBUF + b
            drain(b)
            accs = lax.fori_loop(0, K, make_edge(b), accs)

            @pl.when(c + NBUF < CHUNKS)
            def _():
                issue(c + NBUF, b)

        return accs

    zero = jnp.zeros((L,), jnp.float32)
    accs = tuple(zero for _ in range(2 * G))
    accs = lax.fori_loop(0, CHUNKS // NBUF, outer, accs)
    for t in range((CHUNKS // NBUF) * NBUF, CHUNKS):
        b = t % NBUF
        drain(b)
        accs = lax.fori_loop(0, K, make_edge(b), accs)

    for g in range(G):
        stat_v[0, pl.ds(g * L, L)] = accs[g]
        stat_v[1, pl.ds(g * L, L)] = accs[G + g]
    pltpu.sync_copy(stat_v, out_hbm.at[wid])


KB = 40
CHUNKS_B = EPW // KB
NIDX = 4


@functools.partial(
    pl.kernel,
    mesh=_mesh,
    out_type=jax.ShapeDtypeStruct((NC, NPAD, C), jnp.float32),
    scratch_types=[
        pltpu.VMEM((NIDX, 2, KB), jnp.int32),
        pltpu.VMEM((NAB, KB), jnp.int32),
        pltpu.VMEM((NBUF, KB, C), jnp.float32),
        pltpu.VMEM((NBUF, KB, C), jnp.float32),
        pltpu.VMEM((NAB, KB, C), jnp.float32),
        pltpu.VMEM_SHARED((NPAD, C), jnp.float32),
        pltpu.SemaphoreType.DMA,
        pltpu.SemaphoreType.DMA,
        pltpu.SemaphoreType.DMA,
    ],
)
def _edge_kernel(row_hbm, col_hbm, p_hbm, q_hbm, out_hbm,
                 idx, scat_idx, buf_p, buf_q, act, accum,
                 sem_i, sem_g, sem_s):
    cid = lax.axis_index("c")
    sid = lax.axis_index("s")
    wid = sid * NC + cid

    zero = jnp.zeros((L,), jnp.float32)

    def zrow(j, _):
        for g in range(G):
            act[0, j, pl.ds(g * L, L)] = zero
        return 0

    lax.fori_loop(0, KB, zrow, 0)
    for rep in range(ROWS_PER_TILE // KB):
        pltpu.sync_copy(
            act.at[0], accum.at[pl.ds(sid * ROWS_PER_TILE + rep * KB, KB)]
        )
    plsc.subcore_barrier()

    def issue_idx(c, ib):
        pltpu.async_copy(row_hbm.at[wid, c], idx.at[ib, 0], sem_i)
        pltpu.async_copy(col_hbm.at[wid, c], idx.at[ib, 1], sem_i)

    def drain_idx():
        pltpu.make_async_copy(row_hbm.at[0, 0], idx.at[0, 0], sem_i).wait()
        pltpu.make_async_copy(col_hbm.at[0, 0], idx.at[0, 1], sem_i).wait()

    def issue_gather(ib, b):
        pltpu.async_copy(p_hbm.at[idx.at[ib, 0]], buf_p.at[b], sem_g)
        pltpu.async_copy(q_hbm.at[idx.at[ib, 1]], buf_q.at[b], sem_g)

    def drain_gather(b):
        pltpu.make_async_copy(p_hbm.at[idx.at[0, 0]], buf_p.at[b], sem_g).wait()
        pltpu.make_async_copy(q_hbm.at[idx.at[0, 1]], buf_q.at[b], sem_g).wait()

    def drain_scatter(ab):
        pltpu.make_async_copy(
            act.at[ab], accum.at[scat_idx.at[0]], sem_s
        ).wait()

    def make_edge(b, ab):
        def edge(j, _):
            for g in range(G):
                sl = pl.ds(g * L, L)
                y = buf_p[b, j, sl] + buf_q[b, j, sl]
                act[ab, j, sl] = y / (1.0 + jnp.exp(-y))
            return 0

        return edge

    for c0 in range(NIDX):
        issue_idx(c0, c0)
    drain_idx()
    issue_gather(0, 0)
    drain_idx()
    issue_gather(1, 1)

    def body(c, b, ib, ib2, ab):
        drain_gather(b)

        @pl.when(c >= NAB)
        def _():
            drain_scatter(ab)

        lax.fori_loop(0, KB, make_edge(b, ab), 0)
        for o in (0, 16, 24):
            scat_idx[ab, pl.ds(o, L)] = idx[ib, 0, pl.ds(o, L)]
        pltpu.async_copy(act.at[ab], accum.at[scat_idx.at[ab]], sem_s,
                         add=True)

        @pl.when(c + NBUF < CHUNKS_B)
        def _():
            drain_idx()
            issue_gather(ib2, b)

        @pl.when(c + NIDX < CHUNKS_B)
        def _():
            issue_idx(c + NIDX, ib)

    def outer(i, _):
        for u in range(NIDX):
            c = i * NIDX + u
            body(c, u % NBUF, u, (u + NBUF) % NIDX, u % NAB)
        return 0

    lax.fori_loop(0, CHUNKS_B // NIDX, outer, 0)
    for t in range((CHUNKS_B // NIDX) * NIDX, CHUNKS_B):
        body(t, t % NBUF, t % NIDX, (t + NBUF) % NIDX, t % NAB)
    for ab in range(NAB):
        drain_scatter(ab)

    plsc.subcore_barrier()
    pltpu.sync_copy(
        accum.at[pl.ds(sid * ROWS_PER_TILE, ROWS_PER_TILE)],
        out_hbm.at[cid, pl.ds(sid * ROWS_PER_TILE, ROWS_PER_TILE)],
    )


def _fold_body(part_ref, out_ref):
    out_ref[...] = part_ref[0] + part_ref[1]


def _fold(partials):
    blk = 1000
    return pl.pallas_call(
        _fold_body,
        grid=(N // blk,),
        in_specs=[pl.BlockSpec((NC, blk, C), lambda i: (0, i, 0))],
        out_specs=pl.BlockSpec((blk, C), lambda i: (i, 0)),
        out_shape=jax.ShapeDtypeStruct((N, C), jnp.float32),
    )(partials)


def kernel(x_bk_c, edge_index_batched, W, gamma, beta):
    row3a = edge_index_batched[0].reshape(NW, CHUNKS, K)
    col3a = edge_index_batched[1].reshape(NW, CHUNKS, K)
    row3b = edge_index_batched[0].reshape(NW, CHUNKS_B, KB)
    col3b = edge_index_batched[1].reshape(NW, CHUNKS_B, KB)
    p_tab, q_tab = _make_pq(x_bk_c, W)

    stats = _stats_kernel(row3a, col3a, p_tab, q_tab)
    tot = jnp.sum(stats, axis=0)
    mean = tot[0] / E
    var = tot[1] / E - mean * mean
    scale = gamma * lax.rsqrt(var + BN_EPS)
    shift = beta - mean * scale
    ss = jnp.stack([scale, shift])
    p2, q2 = _ssfold(p_tab, q_tab, ss)

    partials = _edge_kernel(row3b, col3b, p2, q2)
    return _fold(partials)


# speedup vs baseline: 7.1724x; 1.0701x over previous
import functools

import jax
import jax.numpy as jnp
from jax import lax
from jax.experimental import pallas as pl
from jax.experimental.pallas import tpu as pltpu
from jax.experimental.pallas import tpu_sc as plsc

N = 10000
E = 320000
C = 128
BN_EPS = 1e-5

NC, NS, L = 2, 16, 16
NW = NC * NS
EPW = E // NW
G = C // L
NPAD = 10240
ROWS_PER_TILE = NPAD // NS

KA = 80
CHUNKS_A = EPW // KA
NBUF_A = 4

KB = 40
CHUNKS_B = EPW // KB
NBUF = 2
NAB = 2
NIDX = 4

_mesh = plsc.VectorSubcoreMesh(core_axis_name="c", subcore_axis_name="s")


def _wid():
    return lax.axis_index("s") * NC + lax.axis_index("c")


def _pq_body(x_ref, w_ref, p_ref, q_ref):
    wt = w_ref[0:C, :]
    wb = w_ref[C : 2 * C, :]
    xb = x_ref[...]
    p_ref[...] = jnp.dot(xb, wt - wb, preferred_element_type=jnp.float32)
    q_ref[...] = jnp.dot(xb, wb, preferred_element_type=jnp.float32)


def _make_pq(x, w):
    blk = 1000
    return pl.pallas_call(
        _pq_body,
        grid=(N // blk,),
        in_specs=[
            pl.BlockSpec((blk, C), lambda i: (i, 0)),
            pl.BlockSpec((2 * C, C), lambda i: (0, 0)),
        ],
        out_specs=[
            pl.BlockSpec((blk, C), lambda i: (i, 0)),
            pl.BlockSpec((blk, C), lambda i: (i, 0)),
        ],
        out_shape=[
            jax.ShapeDtypeStruct((N, C), jnp.float32),
            jax.ShapeDtypeStruct((N, C), jnp.float32),
        ],
    )(x, w)


def _ssfold_body(p_ref, q_ref, st_ref, g_ref, b_ref, p2_ref, q2_ref):
    tot = jnp.sum(st_ref[...], axis=0)
    mean = tot[0:1] * (1.0 / E)
    var = tot[1:2] * (1.0 / E) - mean * mean
    scale = g_ref[...] * lax.rsqrt(var + BN_EPS)
    shift = b_ref[...] - mean * scale
    p2_ref[...] = p_ref[...] * scale
    q2_ref[...] = q_ref[...] * scale + shift


def _ssfold(p, q, stats, gamma, beta):
    blk = 1000
    return pl.pallas_call(
        _ssfold_body,
        grid=(N // blk,),
        in_specs=[
            pl.BlockSpec((blk, C), lambda i: (i, 0)),
            pl.BlockSpec((blk, C), lambda i: (i, 0)),
            pl.BlockSpec((NW, 2, C), lambda i: (0, 0, 0)),
            pl.BlockSpec((1, C), lambda i: (0, 0)),
            pl.BlockSpec((1, C), lambda i: (0, 0)),
        ],
        out_specs=[
            pl.BlockSpec((blk, C), lambda i: (i, 0)),
            pl.BlockSpec((blk, C), lambda i: (i, 0)),
        ],
        out_shape=[
            jax.ShapeDtypeStruct((N, C), jnp.float32),
            jax.ShapeDtypeStruct((N, C), jnp.float32),
        ],
    )(p, q, stats, gamma, beta)


@functools.partial(
    pl.kernel,
    mesh=_mesh,
    out_type=jax.ShapeDtypeStruct((NW, 2, C), jnp.float32),
    scratch_types=[
        pltpu.VMEM((CHUNKS_A, KA), jnp.int32),
        pltpu.VMEM((CHUNKS_A, KA), jnp.int32),
        pltpu.VMEM((NBUF_A, KA, C), jnp.float32),
        pltpu.VMEM((NBUF_A, KA, C), jnp.float32),
        pltpu.VMEM((2, C), jnp.float32),
        pltpu.SemaphoreType.DMA,
    ],
)
def _stats_kernel(row_hbm, col_hbm, p_hbm, q_hbm, out_hbm,
                  idx_r, idx_c, buf_p, buf_q, stat_v, sem_g):
    wid = _wid()
    pltpu.sync_copy(row_hbm.at[wid], idx_r)
    pltpu.sync_copy(col_hbm.at[wid], idx_c)

    def issue(c, b):
        pltpu.async_copy(p_hbm.at[idx_r.at[c]], buf_p.at[b], sem_g)
        pltpu.async_copy(q_hbm.at[idx_c.at[c]], buf_q.at[b], sem_g)

    def drain(b):
        pltpu.make_async_copy(p_hbm.at[idx_r.at[0]], buf_p.at[b], sem_g).wait()
        pltpu.make_async_copy(q_hbm.at[idx_c.at[0]], buf_q.at[b], sem_g).wait()

    for b in range(NBUF_A):
        issue(b, b)

    def make_edge(b):
        def edge(j, accs):
            out = []
            for g in range(G):
                sl = pl.ds(g * L, L)
                h = buf_p[b, j, sl] + buf_q[b, j, sl]
                out.append(accs[g] + h)
                out.append(accs[G + g] + h * h)
            return tuple(out[0::2] + out[1::2])

        return edge

    def outer(i, accs):
        for b in range(NBUF_A):
            c = i * NBUF_A + b
            drain(b)
            accs = lax.fori_loop(0, KA, make_edge(b), accs)

            @pl.when(c + NBUF_A < CHUNKS_A)
            def _():
                issue(c + NBUF_A, b)

        return accs

    zero = jnp.zeros((L,), jnp.float32)
    accs = tuple(zero for _ in range(2 * G))
    accs = lax.fori_loop(0, CHUNKS_A // NBUF_A, outer, accs)
    for t in range((CHUNKS_A // NBUF_A) * NBUF_A, CHUNKS_A):
        b = t % NBUF_A
        drain(b)
        accs = lax.fori_loop(0, KA, make_edge(b), accs)

    for g in range(G):
        stat_v[0, pl.ds(g * L, L)] = accs[g]
        stat_v[1, pl.ds(g * L, L)] = accs[G + g]
    pltpu.sync_copy(stat_v, out_hbm.at[wid])


@functools.partial(
    pl.kernel,
    mesh=_mesh,
    out_type=jax.ShapeDtypeStruct((NC, NPAD, C), jnp.float32),
    scratch_types=[
        pltpu.VMEM((NIDX, 2, KB), jnp.int32),
        pltpu.VMEM((NAB, 2 * KB), jnp.int32),
        pltpu.VMEM((NBUF, KB, C), jnp.float32),
        pltpu.VMEM((NBUF, KB, C), jnp.float32),
        pltpu.VMEM((NAB, 2 * KB, C), jnp.float32),
        pltpu.VMEM_SHARED((NPAD, C), jnp.float32),
        pltpu.SemaphoreType.DMA,
        pltpu.SemaphoreType.DMA,
        pltpu.SemaphoreType.DMA,
    ],
)
def _edge_kernel(ei_hbm, p_hbm, q_hbm, out_hbm,
                 idx, scat_idx, buf_p, buf_q, act, accum,
                 sem_i, sem_g, sem_s):
    cid = lax.axis_index("c")
    sid = lax.axis_index("s")
    wid = sid * NC + cid

    zero = jnp.zeros((L,), jnp.float32)

    def zrow(j, _):
        for g in range(G):
            act[0, j, pl.ds(g * L, L)] = zero
        return 0

    lax.fori_loop(0, 2 * KB, zrow, 0)
    for rep in range(ROWS_PER_TILE // (2 * KB)):
        pltpu.sync_copy(
            act.at[0],
            accum.at[pl.ds(sid * ROWS_PER_TILE + rep * 2 * KB, 2 * KB)],
        )
    plsc.subcore_barrier()

    def issue_idx(c, ib):
        pltpu.async_copy(ei_hbm.at[wid, c], idx.at[ib], sem_i)

    def drain_idx():
        pltpu.make_async_copy(ei_hbm.at[0, 0], idx.at[0], sem_i).wait()

    def issue_gather(ib, b):
        pltpu.async_copy(p_hbm.at[idx.at[ib, 0]], buf_p.at[b], sem_g)
        pltpu.async_copy(q_hbm.at[idx.at[ib, 1]], buf_q.at[b], sem_g)

    def drain_gather(b):
        pltpu.make_async_copy(p_hbm.at[idx.at[0, 0]], buf_p.at[b], sem_g).wait()
        pltpu.make_async_copy(q_hbm.at[idx.at[0, 1]], buf_q.at[b], sem_g).wait()

    def drain_scatter(sb):
        pltpu.make_async_copy(
            act.at[sb], accum.at[scat_idx.at[0]], sem_s
        ).wait()

    def make_edge(b, sb, half):
        def edge(j, _):
            for g in range(G):
                sl = pl.ds(g * L, L)
                y = buf_p[b, j, sl] + buf_q[b, j, sl]
                act[sb, half * KB + j, sl] = y / (1.0 + jnp.exp(-y))
            return 0

        return edge

    for c0 in range(NIDX):
        issue_idx(c0, c0)
    drain_idx()
    issue_gather(0, 0)
    drain_idx()
    issue_gather(1, 1)

    def body(c, u):
        b = u % NBUF
        ib = u % NIDX
        ib2 = (u + NBUF) % NIDX
        sb = (u // 2) % NAB
        half = u % 2

        drain_gather(b)
        if half == 0:
            @pl.when(c >= 2 * NAB)
            def _():
                drain_scatter(sb)

        lax.fori_loop(0, KB, make_edge(b, sb, half), 0)
        for o in (0, 16, 24):
            scat_idx[sb, pl.ds(half * KB + o, L)] = idx[ib, 0, pl.ds(o, L)]
        if half == 1:
            pltpu.async_copy(act.at[sb], accum.at[scat_idx.at[sb]], sem_s,
                             add=True)

        @pl.when(c + NBUF < CHUNKS_B)
        def _():
            drain_idx()
            issue_gather(ib2, b)

        @pl.when(c + NIDX < CHUNKS_B)
        def _():
            issue_idx(c + NIDX, ib)

    def outer(i, _):
        for u in range(NIDX):
            body(i * NIDX + u, u)
        return 0

    lax.fori_loop(0, CHUNKS_B // NIDX, outer, 0)
    for t in range((CHUNKS_B // NIDX) * NIDX, CHUNKS_B):
        body(t, t % NIDX)
    for sb in range(NAB):
        drain_scatter(sb)

    plsc.subcore_barrier()
    pltpu.sync_copy(
        accum.at[pl.ds(sid * ROWS_PER_TILE, ROWS_PER_TILE)],
        out_hbm.at[cid, pl.ds(sid * ROWS_PER_TILE, ROWS_PER_TILE)],
    )


def _fold_body(part_ref, out_ref):
    out_ref[...] = part_ref[0] + part_ref[1]


def _fold(partials):
    blk = 1000
    return pl.pallas_call(
        _fold_body,
        grid=(N // blk,),
        in_specs=[pl.BlockSpec((NC, blk, C), lambda i: (0, i, 0))],
        out_specs=pl.BlockSpec((blk, C), lambda i: (i, 0)),
        out_shape=jax.ShapeDtypeStruct((N, C), jnp.float32),
    )(partials)


def kernel(x_bk_c, edge_index_batched, W, gamma, beta):
    row3a = edge_index_batched[0].reshape(NW, CHUNKS_A, KA)
    col3a = edge_index_batched[1].reshape(NW, CHUNKS_A, KA)
    ei4 = jnp.stack(
        [
            edge_index_batched[0].reshape(NW, CHUNKS_B, KB),
            edge_index_batched[1].reshape(NW, CHUNKS_B, KB),
        ],
        axis=2,
    )
    p_tab, q_tab = _make_pq(x_bk_c, W)

    stats = _stats_kernel(row3a, col3a, p_tab, q_tab)
    p2, q2 = _ssfold(p_tab, q_tab, stats,
                     gamma.reshape(1, C), beta.reshape(1, C))

    partials = _edge_kernel(ei4, p2, q2)
    return _fold(partials)
